# baseline (speedup 1.0000x reference)
import dataclasses
import functools

import jax
import jax.numpy as jnp
from jax import lax
from jax.experimental import pallas as pl
from jax.experimental.pallas import tpu as pltpu
from jax.experimental.pallas import tpu_sc as plsc

N = 10000
E = 320000
B = 256
D = 128
NF = 9
V = 119
L = 4
OUT = 10

_NC = 2
_NS = 16
_NW = _NC * _NS
_C = 48
_NCH = 210
_EPT = _NCH * _C
_EP = _NW * _EPT
_DUMMY = N
_NT = 10016
_NPAD = _NT
_DR0 = 632
_DR15 = _NPAD - 15 * _DR0

_mesh = plsc.VectorSubcoreMesh(core_axis_name="c", subcore_axis_name="s")
_cp = pltpu.CompilerParams()
if "needs_layout_passes" in pltpu.CompilerParams.__dataclass_fields__:
    _cp = dataclasses.replace(_cp, needs_layout_passes=False)


@functools.partial(
    pl.kernel,
    out_type=(
        jax.ShapeDtypeStruct((_NC, _NPAD, D), jnp.float32),
        jax.ShapeDtypeStruct((_NW * _NT,), jnp.float32),
    ),
    mesh=_mesh,
    compiler_params=_cp,
    scratch_types=[
        pltpu.VMEM_SHARED((_NPAD, D), jnp.float32),
        pltpu.VMEM((_NT,), jnp.float32),
        pltpu.VMEM((_NT,), jnp.float32),
        pltpu.VMEM((_NT,), jnp.float32),
        pltpu.VMEM((_C,), jnp.int32),
        pltpu.VMEM((_C,), jnp.int32),
        pltpu.VMEM((_C,), jnp.int32),
        pltpu.VMEM((_C,), jnp.int32),
        pltpu.VMEM((_C,), jnp.int32),
        pltpu.VMEM((_C,), jnp.int32),
        pltpu.VMEM((_C, D), jnp.float32),
        pltpu.VMEM((_C, D), jnp.float32),
        pltpu.VMEM((_C, D), jnp.float32),
        pltpu.VMEM((_C,), jnp.float32),
        pltpu.SemaphoreType.DMA,
        pltpu.SemaphoreType.DMA,
        pltpu.SemaphoreType.DMA,
        pltpu.SemaphoreType.DMA,
        pltpu.SemaphoreType.DMA,
        pltpu.SemaphoreType.DMA,
    ],
)
def _edge_kernel(hp_hbm, zeros_hbm, asrc_hbm, adst_hbm, src_hbm, dst_hbm,
                 acc_hbm, spart_hbm,
                 acc_sh, as_v, ad_v, sp_v, src0, src1, src2, dst0, dst1, dst2,
                 rows0, rows1, rows2, e_v, gs0, gs1, gs2, ss0, ss1, ss2):
    core = lax.axis_index("c")
    sub = lax.axis_index("s")
    wid = core * _NS + sub
    ebase = wid * _EPT
    srcb = (src0, src1, src2)
    dstb = (dst0, dst1, dst2)
    rowsb = (rows0, rows1, rows2)
    gsem = (gs0, gs1, gs2)
    ssem = (ss0, ss1, ss2)
    dr_base = sub * _DR0

    pltpu.sync_copy(asrc_hbm, as_v)
    pltpu.sync_copy(adst_hbm, ad_v)

    @pl.when(sub < _NS - 1)
    def _():
        pltpu.sync_copy(zeros_hbm.at[pl.ds(dr_base, _DR0)], acc_sh.at[pl.ds(dr_base, _DR0)])

    @pl.when(sub == _NS - 1)
    def _():
        pltpu.sync_copy(zeros_hbm.at[pl.ds(15 * _DR0, _DR15)], acc_sh.at[pl.ds(15 * _DR0, _DR15)])

    zero16 = jnp.zeros((16,), jnp.float32)

    @pl.loop(0, _NT // 16)
    def _(i):
        sp_v[pl.ds(i * 16, 16)] = zero16

    plsc.subcore_barrier()

    def issue(ci, b, wait_scatter):
        if wait_scatter:
            pltpu.make_async_copy(rowsb[b], acc_sh.at[dstb[b]], ssem[b]).wait()
        off = ebase + ci * _C
        pltpu.sync_copy(src_hbm.at[pl.ds(off, _C)], srcb[b])
        pltpu.sync_copy(dst_hbm.at[pl.ds(off, _C)], dstb[b])
        pltpu.async_copy(hp_hbm.at[srcb[b]], rowsb[b], gsem[b])

    def compute(b):
        src_v, dst_v, rows_v = srcb[b], dstb[b], rowsb[b]
        pltpu.make_async_copy(hp_hbm.at[src_v], rows_v, gsem[b]).wait()
        for j in range(_C // 16):
            s16 = src_v[pl.ds(j * 16, 16)]
            d16 = dst_v[pl.ds(j * 16, 16)]
            z = plsc.load_gather(as_v, [s16]) + plsc.load_gather(ad_v, [d16])
            e = jnp.exp(jnp.where(z >= 0.0, z, 0.2 * z))
            plsc.addupdate_scatter(sp_v, [d16], e)
            e_v[pl.ds(j * 16, 16)] = e

        @pl.loop(0, _C)
        def _(i):
            w = plsc.load_gather(e_v, [jnp.full((16,), 0, jnp.int32) + i])
            for j in range(D // 16):
                rows_v[i, pl.ds(j * 16, 16)] = rows_v[i, pl.ds(j * 16, 16)] * w

        pltpu.async_copy(rows_v, acc_sh.at[dst_v], ssem[b], add=True)

    issue(0, 0, False)
    issue(1, 1, False)
    compute(0)
    issue(2, 2, False)
    compute(1)
    issue(3, 0, True)
    compute(2)
    issue(4, 1, True)

    @pl.loop(3, _NCH - 3, step=3)
    def _(g):
        compute(0)
        issue(g + 2, 2, True)
        compute(1)
        issue(g + 3, 0, True)
        compute(2)
        issue(g + 4, 1, True)

    compute(0)
    issue(_NCH - 1, 2, True)
    compute(1)
    compute(2)

    pltpu.make_async_copy(rowsb[0], acc_sh.at[dstb[0]], ssem[0]).wait()
    pltpu.make_async_copy(rowsb[1], acc_sh.at[dstb[1]], ssem[1]).wait()
    pltpu.make_async_copy(rowsb[2], acc_sh.at[dstb[2]], ssem[2]).wait()

    plsc.subcore_barrier()

    @pl.when(sub < _NS - 1)
    def _():
        pltpu.sync_copy(acc_sh.at[pl.ds(dr_base, _DR0)], acc_hbm.at[core, pl.ds(dr_base, _DR0)])

    @pl.when(sub == _NS - 1)
    def _():
        pltpu.sync_copy(acc_sh.at[pl.ds(15 * _DR0, _DR15)], acc_hbm.at[core, pl.ds(15 * _DR0, _DR15)])

    pltpu.sync_copy(sp_v, spart_hbm.at[pl.ds(wid * _NT, _NT)])


_AC = 8


@functools.partial(
    pl.kernel,
    out_type=jax.ShapeDtypeStruct((N, D), jnp.float32),
    mesh=_mesh,
    compiler_params=_cp,
    scratch_types=[
        pltpu.VMEM((_AC * NF,), jnp.int32),
        pltpu.VMEM((_AC * NF,), jnp.int32),
        pltpu.VMEM((_AC * NF, D), jnp.float32),
        pltpu.VMEM((_AC * NF, D), jnp.float32),
        pltpu.VMEM((_AC, D), jnp.float32),
        pltpu.SemaphoreType.DMA,
        pltpu.SemaphoreType.DMA,
    ],
)
def _atom_kernel(tab_hbm, idx_hbm, h_hbm,
                 idx0, idx1, gr0, gr1, out_v, sem0, sem1):
    core = lax.axis_index("c")
    sub = lax.axis_index("s")
    wid = core * _NS + sub
    base = wid * 320
    nch = jnp.where(wid == _NW - 1, (N - base) // _AC, 320 // _AC)
    idxb = (idx0, idx1)
    grb = (gr0, gr1)
    semb = (sem0, sem1)

    def issue(k, b):
        off = (base + k * _AC) * NF
        pltpu.sync_copy(idx_hbm.at[pl.ds(off, _AC * NF)], idxb[b])
        pltpu.async_copy(tab_hbm.at[idxb[b]], grb[b], semb[b])

    def compute(k, b):
        gr_v = grb[b]
        pltpu.make_async_copy(tab_hbm.at[idxb[b]], gr_v, semb[b]).wait()
        for i in range(_AC):
            for j in range(D // 16):
                sl = pl.ds(j * 16, 16)
                acc = gr_v[i * NF, sl]
                for f in range(1, NF):
                    acc = acc + gr_v[i * NF + f, sl]
                out_v[i, sl] = acc
        pltpu.sync_copy(out_v, h_hbm.at[pl.ds(base + k * _AC, _AC)])

    issue(0, 0)
    issue(1, 1)

    @pl.loop(0, nch, step=2)
    def _(k):
        compute(k, 0)

        @pl.when(k + 2 < nch)
        def _():
            issue(k + 2, 0)

        compute(k + 1, 1)

        @pl.when(k + 3 < nch)
        def _():
            issue(k + 3, 1)


_PR = 320
_PCH = 80
_PB = B * D


@functools.partial(
    pl.kernel,
    out_type=(
        jax.ShapeDtypeStruct((_NW * _PB,), jnp.float32),
        jax.ShapeDtypeStruct((_NW * _PB,), jnp.float32),
        jax.ShapeDtypeStruct((_NW * _PB,), jnp.float32),
    ),
    mesh=_mesh,
    compiler_params=_cp,
    scratch_types=[
        pltpu.VMEM((_PB,), jnp.float32),
        pltpu.VMEM((_PB,), jnp.float32),
        pltpu.VMEM((_PB,), jnp.float32),
        pltpu.VMEM((_PCH, D), jnp.float32),
        pltpu.VMEM((_PCH,), jnp.int32),
    ],
)
def _pool_kernel(h_hbm, batch_hbm, mn_hbm, sm_hbm, mx_hbm,
                 mn_v, sm_v, mx_v, rows_v, bat_v):
    core = lax.axis_index("c")
    sub = lax.axis_index("s")
    wid = core * _NS + sub
    base = wid * _PR
    nch = jnp.where(wid == _NW - 1, (N - base) // _PCH, _PR // _PCH)

    pinf = jnp.full((16,), jnp.inf, jnp.float32)
    ninf = jnp.full((16,), -jnp.inf, jnp.float32)
    zv = jnp.zeros((16,), jnp.float32)

    @pl.loop(0, _PB // 16)
    def _(i):
        mn_v[pl.ds(i * 16, 16)] = pinf
        sm_v[pl.ds(i * 16, 16)] = zv
        mx_v[pl.ds(i * 16, 16)] = ninf

    iota = lax.iota(jnp.int32, 16)

    @pl.loop(0, nch)
    def _(k):
        rb = base + k * _PCH
        pltpu.sync_copy(h_hbm.at[pl.ds(rb, _PCH)], rows_v)
        pltpu.sync_copy(batch_hbm.at[pl.ds(rb, _PCH)], bat_v)

        @pl.loop(0, _PCH)
        def _(r):
            g16 = plsc.load_gather(bat_v, [jnp.full((16,), 0, jnp.int32) + r])
            idx0 = g16 * D + iota
            for j in range(D // 16):
                idx = idx0 + j * 16
                row = rows_v[r, pl.ds(j * 16, 16)]
                plsc.store_scatter(mn_v, [idx], jnp.minimum(plsc.load_gather(mn_v, [idx]), row))
                plsc.store_scatter(mx_v, [idx], jnp.maximum(plsc.load_gather(mx_v, [idx]), row))
                plsc.store_scatter(sm_v, [idx], plsc.load_gather(sm_v, [idx]) + row)

    pltpu.sync_copy(mn_v, mn_hbm.at[pl.ds(wid * _PB, _PB)])
    pltpu.sync_copy(sm_v, sm_hbm.at[pl.ds(wid * _PB, _PB)])
    pltpu.sync_copy(mx_v, mx_hbm.at[pl.ds(wid * _PB, _PB)])


def _final_linear_kernel(z_ref, w_ref, b_ref, o_ref):
    o_ref[...] = z_ref[...] @ w_ref[...] + b_ref[...]


_RB = 1000


def _dense_body(h_ref, w_ref, att2_ref, hp_ref, asd_ref):
    hp = jnp.dot(h_ref[...], w_ref[...], preferred_element_type=jnp.float32)
    hp_ref[...] = hp
    asd_ref[...] = jnp.dot(hp, att2_ref[...], preferred_element_type=jnp.float32)


def _dense_call(h, W, att_src, att_dst):
    att2 = jnp.stack([att_src, att_dst], axis=1)
    return pl.pallas_call(
        _dense_body,
        grid=(N // _RB,),
        in_specs=[
            pl.BlockSpec((_RB, D), lambda i: (i, 0)),
            pl.BlockSpec((D, D), lambda i: (0, 0)),
            pl.BlockSpec((D, 2), lambda i: (0, 0)),
        ],
        out_specs=[
            pl.BlockSpec((_RB, D), lambda i: (i, 0)),
            pl.BlockSpec((_RB, 2), lambda i: (i, 0)),
        ],
        out_shape=[
            jax.ShapeDtypeStruct((N, D), jnp.float32),
            jax.ShapeDtypeStruct((N, 2), jnp.float32),
        ],
    )(h, W, att2)


def _combine_body(acc_ref, sp_ref, b_ref, out_ref):
    s = sp_ref[...].sum(axis=1)
    num = acc_ref[0] + acc_ref[1]
    out_ref[...] = jnp.maximum(num / (s[:, None] + 1e-16) + b_ref[...], 0.0)


def _combine_call(acc, sparts, bias):
    sp = sparts.reshape(_NW, _NT).T
    return pl.pallas_call(
        _combine_body,
        grid=(N // _RB,),
        in_specs=[
            pl.BlockSpec((2, _RB, D), lambda i: (0, i, 0)),
            pl.BlockSpec((_RB, _NW), lambda i: (i, 0)),
            pl.BlockSpec((1, D), lambda i: (0, 0)),
        ],
        out_specs=pl.BlockSpec((_RB, D), lambda i: (i, 0)),
        out_shape=jax.ShapeDtypeStruct((N, D), jnp.float32),
    )(acc, sp, bias[None, :])


def _stat_pool(h, batch_i32, cnt):
    mnp, smp, mxp = _pool_kernel(h, batch_i32)
    mn = mnp.reshape(_NW, B, D).min(axis=0)
    mx = mxp.reshape(_NW, B, D).max(axis=0)
    s = smp.reshape(_NW, B, D).sum(axis=0)
    mean = s / jnp.maximum(cnt, 1.0)[:, None]
    return jnp.concatenate([mn, mean, mx], axis=1)


def kernel(params, x, edge_index, batch):
    pad = jnp.zeros((_EP - E,), jnp.int32)
    src = jnp.concatenate([edge_index[0].astype(jnp.int32), pad])
    dst = jnp.concatenate([edge_index[1].astype(jnp.int32), pad + _DUMMY])
    tpad = jnp.zeros((_NT - N,), jnp.float32)
    zeros_nd = jnp.zeros((_NPAD, D), jnp.float32)
    batch_i32 = batch.astype(jnp.int32)
    bnd = jnp.searchsorted(batch_i32, jnp.arange(B + 1, dtype=jnp.int32))
    cnt = (bnd[1:] - bnd[:-1]).astype(jnp.float32)
    xf = (x.astype(jnp.int32) + jnp.arange(NF, dtype=jnp.int32)[None, :] * V).reshape(-1)
    h = _atom_kernel(params['atom_tables'].reshape(NF * V, D), xf)
    jk = [_stat_pool(h, batch_i32, cnt)]
    for i in range(L):
        W, a_src, a_dst, b = params['W'][i], params['att_src'][i], params['att_dst'][i], params['bias'][i]
        hp, asd = _dense_call(h, W, a_src, a_dst)
        a_s = jnp.concatenate([asd[:, 0], tpad])
        a_d = jnp.concatenate([asd[:, 1], tpad])
        acc, sparts = _edge_kernel(hp, zeros_nd, a_s, a_d, src, dst)
        h = _combine_call(acc, sparts, b)
        jk.append(_stat_pool(h, batch_i32, cnt))
    z = jnp.concatenate(jk, axis=1)
    out = pl.pallas_call(
        _final_linear_kernel,
        out_shape=jax.ShapeDtypeStruct((B, OUT), jnp.float32),
    )(z, params['lin_W'], params['lin_b'][None, :])
    return out


# speedup vs baseline: 1.0647x; 1.0647x over previous
import dataclasses
import functools

import jax
import jax.numpy as jnp
from jax import lax
from jax.experimental import pallas as pl
from jax.experimental.pallas import tpu as pltpu
from jax.experimental.pallas import tpu_sc as plsc

N = 10000
E = 320000
B = 256
D = 128
NF = 9
V = 119
L = 4
OUT = 10

_NC = 2
_NS = 16
_NW = _NC * _NS
_C = 48
_NCH = 210
_EPT = _NCH * _C
_EP = _NW * _EPT
_DUMMY = N
_NT = 10016
_NPAD = _NT
_DR0 = 632
_DR15 = _NPAD - 15 * _DR0

_mesh = plsc.VectorSubcoreMesh(core_axis_name="c", subcore_axis_name="s")
_cp = pltpu.CompilerParams()
if "needs_layout_passes" in pltpu.CompilerParams.__dataclass_fields__:
    _cp = dataclasses.replace(_cp, needs_layout_passes=False)


@functools.partial(
    pl.kernel,
    out_type=(
        jax.ShapeDtypeStruct((_NC, _NPAD, D), jnp.float32),
        jax.ShapeDtypeStruct((_NW * _NT,), jnp.float32),
    ),
    mesh=_mesh,
    compiler_params=_cp,
    scratch_types=[
        pltpu.VMEM_SHARED((_NPAD, D), jnp.float32),
        pltpu.VMEM((_NT,), jnp.float32),
        pltpu.VMEM((_NT,), jnp.float32),
        pltpu.VMEM((_NT,), jnp.float32),
        pltpu.VMEM((_C,), jnp.int32),
        pltpu.VMEM((_C,), jnp.int32),
        pltpu.VMEM((_C,), jnp.int32),
        pltpu.VMEM((_C,), jnp.int32),
        pltpu.VMEM((_C,), jnp.int32),
        pltpu.VMEM((_C,), jnp.int32),
        pltpu.VMEM((_C, D), jnp.float32),
        pltpu.VMEM((_C, D), jnp.float32),
        pltpu.VMEM((_C, D), jnp.float32),
        pltpu.VMEM((_C,), jnp.float32),
        pltpu.SemaphoreType.DMA,
        pltpu.SemaphoreType.DMA,
        pltpu.SemaphoreType.DMA,
        pltpu.SemaphoreType.DMA,
        pltpu.SemaphoreType.DMA,
        pltpu.SemaphoreType.DMA,
    ],
)
def _edge_kernel(hp_hbm, zeros_hbm, asrc_hbm, adst_hbm, src_hbm, dst_hbm,
                 acc_hbm, spart_hbm,
                 acc_sh, as_v, ad_v, sp_v, src0, src1, src2, dst0, dst1, dst2,
                 rows0, rows1, rows2, e_v, gs0, gs1, gs2, ss0, ss1, ss2):
    core = lax.axis_index("c")
    sub = lax.axis_index("s")
    wid = core * _NS + sub
    ebase = wid * _EPT
    srcb = (src0, src1, src2)
    dstb = (dst0, dst1, dst2)
    rowsb = (rows0, rows1, rows2)
    gsem = (gs0, gs1, gs2)
    ssem = (ss0, ss1, ss2)
    dr_base = sub * _DR0

    pltpu.sync_copy(asrc_hbm, as_v)
    pltpu.sync_copy(adst_hbm, ad_v)

    @pl.when(sub < _NS - 1)
    def _():
        pltpu.sync_copy(zeros_hbm.at[pl.ds(dr_base, _DR0)], acc_sh.at[pl.ds(dr_base, _DR0)])

    @pl.when(sub == _NS - 1)
    def _():
        pltpu.sync_copy(zeros_hbm.at[pl.ds(15 * _DR0, _DR15)], acc_sh.at[pl.ds(15 * _DR0, _DR15)])

    zero16 = jnp.zeros((16,), jnp.float32)

    @pl.loop(0, _NT // 16)
    def _(i):
        sp_v[pl.ds(i * 16, 16)] = zero16

    plsc.subcore_barrier()

    def issue(ci, b, wait_scatter):
        if wait_scatter:
            pltpu.make_async_copy(rowsb[b], acc_sh.at[dstb[b]], ssem[b]).wait()
        off = ebase + ci * _C
        pltpu.sync_copy(src_hbm.at[pl.ds(off, _C)], srcb[b])
        pltpu.sync_copy(dst_hbm.at[pl.ds(off, _C)], dstb[b])
        pltpu.async_copy(hp_hbm.at[srcb[b]], rowsb[b], gsem[b])

    def compute(b):
        src_v, dst_v, rows_v = srcb[b], dstb[b], rowsb[b]
        pltpu.make_async_copy(hp_hbm.at[src_v], rows_v, gsem[b]).wait()
        for j in range(_C // 16):
            s16 = src_v[pl.ds(j * 16, 16)]
            d16 = dst_v[pl.ds(j * 16, 16)]
            z = plsc.load_gather(as_v, [s16]) + plsc.load_gather(ad_v, [d16])
            e = jnp.exp(jnp.where(z >= 0.0, z, 0.2 * z))
            plsc.addupdate_scatter(sp_v, [d16], e)
            e_v[pl.ds(j * 16, 16)] = e

        @pl.loop(0, _C, step=4)
        def _(i):
            ws = [plsc.load_gather(e_v, [jnp.full((16,), d, jnp.int32) + i])
                  for d in range(4)]
            for j in range(D // 16):
                for d in range(4):
                    sl = pl.ds(j * 16, 16)
                    rows_v[i + d, sl] = rows_v[i + d, sl] * ws[d]

        pltpu.async_copy(rows_v, acc_sh.at[dst_v], ssem[b], add=True)

    issue(0, 0, False)
    issue(1, 1, False)
    compute(0)
    issue(2, 2, False)
    compute(1)
    issue(3, 0, True)
    compute(2)
    issue(4, 1, True)

    @pl.loop(3, _NCH - 3, step=3)
    def _(g):
        compute(0)
        issue(g + 2, 2, True)
        compute(1)
        issue(g + 3, 0, True)
        compute(2)
        issue(g + 4, 1, True)

    compute(0)
    issue(_NCH - 1, 2, True)
    compute(1)
    compute(2)

    pltpu.make_async_copy(rowsb[0], acc_sh.at[dstb[0]], ssem[0]).wait()
    pltpu.make_async_copy(rowsb[1], acc_sh.at[dstb[1]], ssem[1]).wait()
    pltpu.make_async_copy(rowsb[2], acc_sh.at[dstb[2]], ssem[2]).wait()

    plsc.subcore_barrier()

    @pl.when(sub < _NS - 1)
    def _():
        pltpu.sync_copy(acc_sh.at[pl.ds(dr_base, _DR0)], acc_hbm.at[core, pl.ds(dr_base, _DR0)])

    @pl.when(sub == _NS - 1)
    def _():
        pltpu.sync_copy(acc_sh.at[pl.ds(15 * _DR0, _DR15)], acc_hbm.at[core, pl.ds(15 * _DR0, _DR15)])

    pltpu.sync_copy(sp_v, spart_hbm.at[pl.ds(wid * _NT, _NT)])


_AC = 8


@functools.partial(
    pl.kernel,
    out_type=jax.ShapeDtypeStruct((N, D), jnp.float32),
    mesh=_mesh,
    compiler_params=_cp,
    scratch_types=[
        pltpu.VMEM((_AC * NF,), jnp.int32),
        pltpu.VMEM((_AC * NF,), jnp.int32),
        pltpu.VMEM((_AC * NF, D), jnp.float32),
        pltpu.VMEM((_AC * NF, D), jnp.float32),
        pltpu.VMEM((_AC, D), jnp.float32),
        pltpu.SemaphoreType.DMA,
        pltpu.SemaphoreType.DMA,
    ],
)
def _atom_kernel(tab_hbm, idx_hbm, h_hbm,
                 idx0, idx1, gr0, gr1, out_v, sem0, sem1):
    core = lax.axis_index("c")
    sub = lax.axis_index("s")
    wid = core * _NS + sub
    base = wid * 320
    nch = jnp.where(wid == _NW - 1, (N - base) // _AC, 320 // _AC)
    idxb = (idx0, idx1)
    grb = (gr0, gr1)
    semb = (sem0, sem1)

    def issue(k, b):
        off = (base + k * _AC) * NF
        pltpu.sync_copy(idx_hbm.at[pl.ds(off, _AC * NF)], idxb[b])
        pltpu.async_copy(tab_hbm.at[idxb[b]], grb[b], semb[b])

    def compute(k, b):
        gr_v = grb[b]
        pltpu.make_async_copy(tab_hbm.at[idxb[b]], gr_v, semb[b]).wait()
        for i in range(_AC):
            for j in range(D // 16):
                sl = pl.ds(j * 16, 16)
                acc = gr_v[i * NF, sl]
                for f in range(1, NF):
                    acc = acc + gr_v[i * NF + f, sl]
                out_v[i, sl] = acc
        pltpu.sync_copy(out_v, h_hbm.at[pl.ds(base + k * _AC, _AC)])

    issue(0, 0)
    issue(1, 1)

    @pl.loop(0, nch, step=2)
    def _(k):
        compute(k, 0)

        @pl.when(k + 2 < nch)
        def _():
            issue(k + 2, 0)

        compute(k + 1, 1)

        @pl.when(k + 3 < nch)
        def _():
            issue(k + 3, 1)


_PR = 320
_PCH = 80
_PB = B * D


@functools.partial(
    pl.kernel,
    out_type=(
        jax.ShapeDtypeStruct((_NW * _PB,), jnp.float32),
        jax.ShapeDtypeStruct((_NW * _PB,), jnp.float32),
        jax.ShapeDtypeStruct((_NW * _PB,), jnp.float32),
    ),
    mesh=_mesh,
    compiler_params=_cp,
    scratch_types=[
        pltpu.VMEM((_PB,), jnp.float32),
        pltpu.VMEM((_PB,), jnp.float32),
        pltpu.VMEM((_PB,), jnp.float32),
        pltpu.VMEM((_PCH, D), jnp.float32),
        pltpu.VMEM((_PCH,), jnp.int32),
    ],
)
def _pool_kernel(h_hbm, batch_hbm, mn_hbm, sm_hbm, mx_hbm,
                 mn_v, sm_v, mx_v, rows_v, bat_v):
    core = lax.axis_index("c")
    sub = lax.axis_index("s")
    wid = core * _NS + sub
    base = wid * _PR
    nch = jnp.where(wid == _NW - 1, (N - base) // _PCH, _PR // _PCH)

    pinf = jnp.full((16,), jnp.inf, jnp.float32)
    ninf = jnp.full((16,), -jnp.inf, jnp.float32)
    zv = jnp.zeros((16,), jnp.float32)

    @pl.loop(0, _PB // 16)
    def _(i):
        mn_v[pl.ds(i * 16, 16)] = pinf
        sm_v[pl.ds(i * 16, 16)] = zv
        mx_v[pl.ds(i * 16, 16)] = ninf

    iota = lax.iota(jnp.int32, 16)

    @pl.loop(0, nch)
    def _(k):
        rb = base + k * _PCH
        pltpu.sync_copy(h_hbm.at[pl.ds(rb, _PCH)], rows_v)
        pltpu.sync_copy(batch_hbm.at[pl.ds(rb, _PCH)], bat_v)

        @pl.loop(0, _PCH)
        def _(r):
            g16 = plsc.load_gather(bat_v, [jnp.full((16,), 0, jnp.int32) + r])
            idx0 = g16 * D + iota
            for j in range(D // 16):
                idx = idx0 + j * 16
                row = rows_v[r, pl.ds(j * 16, 16)]
                plsc.store_scatter(mn_v, [idx], jnp.minimum(plsc.load_gather(mn_v, [idx]), row))
                plsc.store_scatter(mx_v, [idx], jnp.maximum(plsc.load_gather(mx_v, [idx]), row))
                plsc.store_scatter(sm_v, [idx], plsc.load_gather(sm_v, [idx]) + row)

    pltpu.sync_copy(mn_v, mn_hbm.at[pl.ds(wid * _PB, _PB)])
    pltpu.sync_copy(sm_v, sm_hbm.at[pl.ds(wid * _PB, _PB)])
    pltpu.sync_copy(mx_v, mx_hbm.at[pl.ds(wid * _PB, _PB)])


def _final_linear_kernel(z_ref, w_ref, b_ref, o_ref):
    o_ref[...] = z_ref[...] @ w_ref[...] + b_ref[...]


_RB = 1000


def _dense_body(h_ref, w_ref, att2_ref, hp_ref, asd_ref):
    hp = jnp.dot(h_ref[...], w_ref[...], preferred_element_type=jnp.float32)
    hp_ref[...] = hp
    asd_ref[...] = jnp.dot(hp, att2_ref[...], preferred_element_type=jnp.float32)


def _dense_call(h, W, att_src, att_dst):
    att2 = jnp.stack([att_src, att_dst], axis=1)
    return pl.pallas_call(
        _dense_body,
        grid=(N // _RB,),
        in_specs=[
            pl.BlockSpec((_RB, D), lambda i: (i, 0)),
            pl.BlockSpec((D, D), lambda i: (0, 0)),
            pl.BlockSpec((D, 2), lambda i: (0, 0)),
        ],
        out_specs=[
            pl.BlockSpec((_RB, D), lambda i: (i, 0)),
            pl.BlockSpec((_RB, 2), lambda i: (i, 0)),
        ],
        out_shape=[
            jax.ShapeDtypeStruct((N, D), jnp.float32),
            jax.ShapeDtypeStruct((N, 2), jnp.float32),
        ],
    )(h, W, att2)


def _combine_body(acc_ref, sp_ref, b_ref, out_ref):
    s = sp_ref[...].sum(axis=1)
    num = acc_ref[0] + acc_ref[1]
    out_ref[...] = jnp.maximum(num / (s[:, None] + 1e-16) + b_ref[...], 0.0)


def _combine_call(acc, sparts, bias):
    sp = sparts.reshape(_NW, _NT).T
    return pl.pallas_call(
        _combine_body,
        grid=(N // _RB,),
        in_specs=[
            pl.BlockSpec((2, _RB, D), lambda i: (0, i, 0)),
            pl.BlockSpec((_RB, _NW), lambda i: (i, 0)),
            pl.BlockSpec((1, D), lambda i: (0, 0)),
        ],
        out_specs=pl.BlockSpec((_RB, D), lambda i: (i, 0)),
        out_shape=jax.ShapeDtypeStruct((N, D), jnp.float32),
    )(acc, sp, bias[None, :])


def _stat_pool(h, batch_i32, cnt):
    mnp, smp, mxp = _pool_kernel(h, batch_i32)
    mn = mnp.reshape(_NW, B, D).min(axis=0)
    mx = mxp.reshape(_NW, B, D).max(axis=0)
    s = smp.reshape(_NW, B, D).sum(axis=0)
    mean = s / jnp.maximum(cnt, 1.0)[:, None]
    return jnp.concatenate([mn, mean, mx], axis=1)


def kernel(params, x, edge_index, batch):
    pad = jnp.zeros((_EP - E,), jnp.int32)
    src = jnp.concatenate([edge_index[0].astype(jnp.int32), pad])
    dst = jnp.concatenate([edge_index[1].astype(jnp.int32), pad + _DUMMY])
    tpad = jnp.zeros((_NT - N,), jnp.float32)
    zeros_nd = jnp.zeros((_NPAD, D), jnp.float32)
    batch_i32 = batch.astype(jnp.int32)
    bnd = jnp.searchsorted(batch_i32, jnp.arange(B + 1, dtype=jnp.int32))
    cnt = (bnd[1:] - bnd[:-1]).astype(jnp.float32)
    xf = (x.astype(jnp.int32) + jnp.arange(NF, dtype=jnp.int32)[None, :] * V).reshape(-1)
    h = _atom_kernel(params['atom_tables'].reshape(NF * V, D), xf)
    jk = [_stat_pool(h, batch_i32, cnt)]
    for i in range(L):
        W, a_src, a_dst, b = params['W'][i], params['att_src'][i], params['att_dst'][i], params['bias'][i]
        hp, asd = _dense_call(h, W, a_src, a_dst)
        a_s = jnp.concatenate([asd[:, 0], tpad])
        a_d = jnp.concatenate([asd[:, 1], tpad])
        acc, sparts = _edge_kernel(hp, zeros_nd, a_s, a_d, src, dst)
        h = _combine_call(acc, sparts, b)
        jk.append(_stat_pool(h, batch_i32, cnt))
    z = jnp.concatenate(jk, axis=1)
    out = pl.pallas_call(
        _final_linear_kernel,
        out_shape=jax.ShapeDtypeStruct((B, OUT), jnp.float32),
    )(z, params['lin_W'], params['lin_b'][None, :])
    return out


# speedup vs baseline: 1.0733x; 1.0081x over previous
import dataclasses
import functools

import jax
import jax.numpy as jnp
from jax import lax
from jax.experimental import pallas as pl
from jax.experimental.pallas import tpu as pltpu
from jax.experimental.pallas import tpu_sc as plsc

N = 10000
E = 320000
B = 256
D = 128
NF = 9
V = 119
L = 4
OUT = 10

_NC = 2
_NS = 16
_NW = _NC * _NS
_C = 48
_NCH = 210
_EPT = _NCH * _C
_EP = _NW * _EPT
_DUMMY = N
_NT = 10016
_NPAD = _NT
_DR0 = 632
_DR15 = _NPAD - 15 * _DR0

_mesh = plsc.VectorSubcoreMesh(core_axis_name="c", subcore_axis_name="s")
_cp = pltpu.CompilerParams()
if "needs_layout_passes" in pltpu.CompilerParams.__dataclass_fields__:
    _cp = dataclasses.replace(_cp, needs_layout_passes=False)


@functools.partial(
    pl.kernel,
    out_type=(
        jax.ShapeDtypeStruct((_NC, _NPAD, D), jnp.float32),
        jax.ShapeDtypeStruct((_NW * _NT,), jnp.float32),
    ),
    mesh=_mesh,
    compiler_params=_cp,
    scratch_types=[
        pltpu.VMEM_SHARED((_NPAD, D), jnp.float32),
        pltpu.VMEM((_NT,), jnp.float32),
        pltpu.VMEM((_NT,), jnp.float32),
        pltpu.VMEM((_NT,), jnp.float32),
        pltpu.VMEM((_C,), jnp.int32),
        pltpu.VMEM((_C,), jnp.int32),
        pltpu.VMEM((_C,), jnp.int32),
        pltpu.VMEM((_C,), jnp.int32),
        pltpu.VMEM((_C,), jnp.int32),
        pltpu.VMEM((_C,), jnp.int32),
        pltpu.VMEM((_C, D), jnp.float32),
        pltpu.VMEM((_C, D), jnp.float32),
        pltpu.VMEM((_C, D), jnp.float32),
        pltpu.VMEM((_C,), jnp.float32),
        pltpu.SemaphoreType.DMA,
        pltpu.SemaphoreType.DMA,
        pltpu.SemaphoreType.DMA,
        pltpu.SemaphoreType.DMA,
        pltpu.SemaphoreType.DMA,
        pltpu.SemaphoreType.DMA,
    ],
)
def _edge_kernel(hp_hbm, zeros_hbm, asrc_hbm, adst_hbm, src_hbm, dst_hbm,
                 acc_hbm, spart_hbm,
                 acc_sh, as_v, ad_v, sp_v, src0, src1, src2, dst0, dst1, dst2,
                 rows0, rows1, rows2, e_v, gs0, gs1, gs2, ss0, ss1, ss2):
    core = lax.axis_index("c")
    sub = lax.axis_index("s")
    wid = core * _NS + sub
    ebase = wid * _EPT
    srcb = (src0, src1, src2)
    dstb = (dst0, dst1, dst2)
    rowsb = (rows0, rows1, rows2)
    gsem = (gs0, gs1, gs2)
    ssem = (ss0, ss1, ss2)
    dr_base = sub * _DR0

    pltpu.sync_copy(asrc_hbm, as_v)
    pltpu.sync_copy(adst_hbm, ad_v)

    @pl.when(sub < _NS - 1)
    def _():
        pltpu.sync_copy(zeros_hbm.at[pl.ds(dr_base, _DR0)], acc_sh.at[pl.ds(dr_base, _DR0)])

    @pl.when(sub == _NS - 1)
    def _():
        pltpu.sync_copy(zeros_hbm.at[pl.ds(15 * _DR0, _DR15)], acc_sh.at[pl.ds(15 * _DR0, _DR15)])

    zero16 = jnp.zeros((16,), jnp.float32)

    @pl.loop(0, _NT // 16)
    def _(i):
        sp_v[pl.ds(i * 16, 16)] = zero16

    plsc.subcore_barrier()

    def issue(ci, b, wait_scatter):
        if wait_scatter:
            pltpu.make_async_copy(rowsb[b], acc_sh.at[dstb[b]], ssem[b]).wait()
        off = ebase + ci * _C
        pltpu.sync_copy(src_hbm.at[pl.ds(off, _C)], srcb[b])
        pltpu.sync_copy(dst_hbm.at[pl.ds(off, _C)], dstb[b])
        pltpu.async_copy(hp_hbm.at[srcb[b]], rowsb[b], gsem[b])

    def compute(b):
        src_v, dst_v, rows_v = srcb[b], dstb[b], rowsb[b]
        pltpu.make_async_copy(hp_hbm.at[src_v], rows_v, gsem[b]).wait()
        for j in range(_C // 16):
            s16 = src_v[pl.ds(j * 16, 16)]
            d16 = dst_v[pl.ds(j * 16, 16)]
            z = plsc.load_gather(as_v, [s16]) + plsc.load_gather(ad_v, [d16])
            e = jnp.exp(jnp.where(z >= 0.0, z, 0.2 * z))
            plsc.addupdate_scatter(sp_v, [d16], e)
            e_v[pl.ds(j * 16, 16)] = e

        @plsc.parallel_loop(0, _C, unroll=4)
        def _(i):
            w = plsc.load_gather(e_v, [jnp.full((16,), 0, jnp.int32) + i])
            for j in range(D // 16):
                sl = pl.ds(j * 16, 16)
                rows_v[i, sl] = rows_v[i, sl] * w

        pltpu.async_copy(rows_v, acc_sh.at[dst_v], ssem[b], add=True)

    issue(0, 0, False)
    issue(1, 1, False)
    compute(0)
    issue(2, 2, False)
    compute(1)
    issue(3, 0, True)
    compute(2)
    issue(4, 1, True)

    @pl.loop(3, _NCH - 3, step=3)
    def _(g):
        compute(0)
        issue(g + 2, 2, True)
        compute(1)
        issue(g + 3, 0, True)
        compute(2)
        issue(g + 4, 1, True)

    compute(0)
    issue(_NCH - 1, 2, True)
    compute(1)
    compute(2)

    pltpu.make_async_copy(rowsb[0], acc_sh.at[dstb[0]], ssem[0]).wait()
    pltpu.make_async_copy(rowsb[1], acc_sh.at[dstb[1]], ssem[1]).wait()
    pltpu.make_async_copy(rowsb[2], acc_sh.at[dstb[2]], ssem[2]).wait()

    plsc.subcore_barrier()

    @pl.when(sub < _NS - 1)
    def _():
        pltpu.sync_copy(acc_sh.at[pl.ds(dr_base, _DR0)], acc_hbm.at[core, pl.ds(dr_base, _DR0)])

    @pl.when(sub == _NS - 1)
    def _():
        pltpu.sync_copy(acc_sh.at[pl.ds(15 * _DR0, _DR15)], acc_hbm.at[core, pl.ds(15 * _DR0, _DR15)])

    pltpu.sync_copy(sp_v, spart_hbm.at[pl.ds(wid * _NT, _NT)])


_AC = 8


@functools.partial(
    pl.kernel,
    out_type=jax.ShapeDtypeStruct((N, D), jnp.float32),
    mesh=_mesh,
    compiler_params=_cp,
    scratch_types=[
        pltpu.VMEM((_AC * NF,), jnp.int32),
        pltpu.VMEM((_AC * NF,), jnp.int32),
        pltpu.VMEM((_AC * NF, D), jnp.float32),
        pltpu.VMEM((_AC * NF, D), jnp.float32),
        pltpu.VMEM((_AC, D), jnp.float32),
        pltpu.SemaphoreType.DMA,
        pltpu.SemaphoreType.DMA,
    ],
)
def _atom_kernel(tab_hbm, idx_hbm, h_hbm,
                 idx0, idx1, gr0, gr1, out_v, sem0, sem1):
    core = lax.axis_index("c")
    sub = lax.axis_index("s")
    wid = core * _NS + sub
    base = wid * 320
    nch = jnp.where(wid == _NW - 1, (N - base) // _AC, 320 // _AC)
    idxb = (idx0, idx1)
    grb = (gr0, gr1)
    semb = (sem0, sem1)

    def issue(k, b):
        off = (base + k * _AC) * NF
        pltpu.sync_copy(idx_hbm.at[pl.ds(off, _AC * NF)], idxb[b])
        pltpu.async_copy(tab_hbm.at[idxb[b]], grb[b], semb[b])

    def compute(k, b):
        gr_v = grb[b]
        pltpu.make_async_copy(tab_hbm.at[idxb[b]], gr_v, semb[b]).wait()
        for i in range(_AC):
            for j in range(D // 16):
                sl = pl.ds(j * 16, 16)
                acc = gr_v[i * NF, sl]
                for f in range(1, NF):
                    acc = acc + gr_v[i * NF + f, sl]
                out_v[i, sl] = acc
        pltpu.sync_copy(out_v, h_hbm.at[pl.ds(base + k * _AC, _AC)])

    issue(0, 0)
    issue(1, 1)

    @pl.loop(0, nch, step=2)
    def _(k):
        compute(k, 0)

        @pl.when(k + 2 < nch)
        def _():
            issue(k + 2, 0)

        compute(k + 1, 1)

        @pl.when(k + 3 < nch)
        def _():
            issue(k + 3, 1)


_PR = 320
_PCH = 80
_PB = B * D


@functools.partial(
    pl.kernel,
    out_type=(
        jax.ShapeDtypeStruct((_NW * _PB,), jnp.float32),
        jax.ShapeDtypeStruct((_NW * _PB,), jnp.float32),
        jax.ShapeDtypeStruct((_NW * _PB,), jnp.float32),
    ),
    mesh=_mesh,
    compiler_params=_cp,
    scratch_types=[
        pltpu.VMEM((_PB,), jnp.float32),
        pltpu.VMEM((_PB,), jnp.float32),
        pltpu.VMEM((_PB,), jnp.float32),
        pltpu.VMEM((_PCH, D), jnp.float32),
        pltpu.VMEM((_PCH,), jnp.int32),
    ],
)
def _pool_kernel(h_hbm, batch_hbm, mn_hbm, sm_hbm, mx_hbm,
                 mn_v, sm_v, mx_v, rows_v, bat_v):
    core = lax.axis_index("c")
    sub = lax.axis_index("s")
    wid = core * _NS + sub
    base = wid * _PR
    nch = jnp.where(wid == _NW - 1, (N - base) // _PCH, _PR // _PCH)

    pinf = jnp.full((16,), jnp.inf, jnp.float32)
    ninf = jnp.full((16,), -jnp.inf, jnp.float32)
    zv = jnp.zeros((16,), jnp.float32)

    @pl.loop(0, _PB // 16)
    def _(i):
        mn_v[pl.ds(i * 16, 16)] = pinf
        sm_v[pl.ds(i * 16, 16)] = zv
        mx_v[pl.ds(i * 16, 16)] = ninf

    iota = lax.iota(jnp.int32, 16)

    @pl.loop(0, nch)
    def _(k):
        rb = base + k * _PCH
        pltpu.sync_copy(h_hbm.at[pl.ds(rb, _PCH)], rows_v)
        pltpu.sync_copy(batch_hbm.at[pl.ds(rb, _PCH)], bat_v)

        @pl.loop(0, _PCH)
        def _(r):
            g16 = plsc.load_gather(bat_v, [jnp.full((16,), 0, jnp.int32) + r])
            idx0 = g16 * D + iota
            for j in range(D // 16):
                idx = idx0 + j * 16
                row = rows_v[r, pl.ds(j * 16, 16)]
                plsc.store_scatter(mn_v, [idx], jnp.minimum(plsc.load_gather(mn_v, [idx]), row))
                plsc.store_scatter(mx_v, [idx], jnp.maximum(plsc.load_gather(mx_v, [idx]), row))
                plsc.store_scatter(sm_v, [idx], plsc.load_gather(sm_v, [idx]) + row)

    pltpu.sync_copy(mn_v, mn_hbm.at[pl.ds(wid * _PB, _PB)])
    pltpu.sync_copy(sm_v, sm_hbm.at[pl.ds(wid * _PB, _PB)])
    pltpu.sync_copy(mx_v, mx_hbm.at[pl.ds(wid * _PB, _PB)])


def _final_linear_kernel(z_ref, w_ref, b_ref, o_ref):
    o_ref[...] = z_ref[...] @ w_ref[...] + b_ref[...]


_RB = 1000


def _dense_body(h_ref, w_ref, att2_ref, hp_ref, asd_ref):
    hp = jnp.dot(h_ref[...], w_ref[...], preferred_element_type=jnp.float32)
    hp_ref[...] = hp
    asd_ref[...] = jnp.dot(hp, att2_ref[...], preferred_element_type=jnp.float32)


def _dense_call(h, W, att_src, att_dst):
    att2 = jnp.stack([att_src, att_dst], axis=1)
    return pl.pallas_call(
        _dense_body,
        grid=(N // _RB,),
        in_specs=[
            pl.BlockSpec((_RB, D), lambda i: (i, 0)),
            pl.BlockSpec((D, D), lambda i: (0, 0)),
            pl.BlockSpec((D, 2), lambda i: (0, 0)),
        ],
        out_specs=[
            pl.BlockSpec((_RB, D), lambda i: (i, 0)),
            pl.BlockSpec((_RB, 2), lambda i: (i, 0)),
        ],
        out_shape=[
            jax.ShapeDtypeStruct((N, D), jnp.float32),
            jax.ShapeDtypeStruct((N, 2), jnp.float32),
        ],
    )(h, W, att2)


def _combine_body(acc_ref, sp_ref, b_ref, out_ref):
    s = sp_ref[...].sum(axis=1)
    num = acc_ref[0] + acc_ref[1]
    out_ref[...] = jnp.maximum(num / (s[:, None] + 1e-16) + b_ref[...], 0.0)


def _combine_call(acc, sparts, bias):
    sp = sparts.reshape(_NW, _NT).T
    return pl.pallas_call(
        _combine_body,
        grid=(N // _RB,),
        in_specs=[
            pl.BlockSpec((2, _RB, D), lambda i: (0, i, 0)),
            pl.BlockSpec((_RB, _NW), lambda i: (i, 0)),
            pl.BlockSpec((1, D), lambda i: (0, 0)),
        ],
        out_specs=pl.BlockSpec((_RB, D), lambda i: (i, 0)),
        out_shape=jax.ShapeDtypeStruct((N, D), jnp.float32),
    )(acc, sp, bias[None, :])


def _stat_pool(h, batch_i32, cnt):
    mnp, smp, mxp = _pool_kernel(h, batch_i32)
    mn = mnp.reshape(_NW, B, D).min(axis=0)
    mx = mxp.reshape(_NW, B, D).max(axis=0)
    s = smp.reshape(_NW, B, D).sum(axis=0)
    mean = s / jnp.maximum(cnt, 1.0)[:, None]
    return jnp.concatenate([mn, mean, mx], axis=1)


def kernel(params, x, edge_index, batch):
    pad = jnp.zeros((_EP - E,), jnp.int32)
    src = jnp.concatenate([edge_index[0].astype(jnp.int32), pad])
    dst = jnp.concatenate([edge_index[1].astype(jnp.int32), pad + _DUMMY])
    tpad = jnp.zeros((_NT - N,), jnp.float32)
    zeros_nd = jnp.zeros((_NPAD, D), jnp.float32)
    batch_i32 = batch.astype(jnp.int32)
    bnd = jnp.searchsorted(batch_i32, jnp.arange(B + 1, dtype=jnp.int32))
    cnt = (bnd[1:] - bnd[:-1]).astype(jnp.float32)
    xf = (x.astype(jnp.int32) + jnp.arange(NF, dtype=jnp.int32)[None, :] * V).reshape(-1)
    h = _atom_kernel(params['atom_tables'].reshape(NF * V, D), xf)
    jk = [_stat_pool(h, batch_i32, cnt)]
    for i in range(L):
        W, a_src, a_dst, b = params['W'][i], params['att_src'][i], params['att_dst'][i], params['bias'][i]
        hp, asd = _dense_call(h, W, a_src, a_dst)
        a_s = jnp.concatenate([asd[:, 0], tpad])
        a_d = jnp.concatenate([asd[:, 1], tpad])
        acc, sparts = _edge_kernel(hp, zeros_nd, a_s, a_d, src, dst)
        h = _combine_call(acc, sparts, b)
        jk.append(_stat_pool(h, batch_i32, cnt))
    z = jnp.concatenate(jk, axis=1)
    out = pl.pallas_call(
        _final_linear_kernel,
        out_shape=jax.ShapeDtypeStruct((B, OUT), jnp.float32),
    )(z, params['lin_W'], params['lin_b'][None, :])
    return out
